# baseline (speedup 1.0000x reference)
import jax
import jax.numpy as jnp
from jax.experimental import pallas as pl

TOP_K = 2
N_EXPERTS = 8
D_MODEL = 1024
TOKENS_PER_BLOCK = 2048


def _router_kernel(x_ref, w_ref, idx_ref, wgt_ref):
    w = w_ref[...]
    dn = (((1,), (1,)), ((), ()))
    logits = jax.lax.dot_general(w, x_ref[...], dn,
                                 preferred_element_type=jnp.float32)

    m = jnp.max(logits, axis=0, keepdims=True)
    u = jnp.exp(logits - m)
    s = u / jnp.sum(u, axis=0, keepdims=True)

    exp_row = jax.lax.broadcasted_iota(jnp.int32, s.shape, 0)
    v1 = jnp.max(s, axis=0, keepdims=True)
    i1 = jnp.min(jnp.where(s == v1, exp_row, N_EXPERTS),
                 axis=0, keepdims=True)
    masked = jnp.where(exp_row == i1, -jnp.inf, s)
    v2 = jnp.max(masked, axis=0, keepdims=True)
    i2 = jnp.min(jnp.where(masked == v2, exp_row, N_EXPERTS),
                 axis=0, keepdims=True)

    denom = (v1 + v2) + 1e-20
    idx_ref[...] = jnp.concatenate([i1, i2], axis=0)
    wgt_ref[...] = jnp.concatenate([v1 / denom, v2 / denom], axis=0)


@jax.jit
def kernel(hidden_states, weight):
    h = hidden_states.shape[-1]
    x = hidden_states.reshape(-1, h).astype(jnp.float32)
    t = x.shape[0]
    w = weight.astype(jnp.float32)
    b = TOKENS_PER_BLOCK
    idx_t, wgt_t = pl.pallas_call(
        _router_kernel,
        grid=(t // b,),
        in_specs=[
            pl.BlockSpec((b, h), lambda i: (i, 0)),
            pl.BlockSpec((N_EXPERTS, h), lambda i: (0, 0)),
        ],
        out_specs=[
            pl.BlockSpec((TOP_K, b), lambda i: (0, i)),
            pl.BlockSpec((TOP_K, b), lambda i: (0, i)),
        ],
        out_shape=[
            jax.ShapeDtypeStruct((TOP_K, t), jnp.int32),
            jax.ShapeDtypeStruct((TOP_K, t), jnp.float32),
        ],
    )(x, w)
    return (idx_t.T, wgt_t.T)
